# baseline (speedup 1.0000x reference)
import functools

import jax
import jax.numpy as jnp
from jax import lax
from jax.experimental import pallas as pl
from jax.experimental.pallas import tpu as pltpu
from jax.experimental.pallas import tpu_sc as plsc

_TB = 40


def _sc_gather(table2, idx2):
    V2, D2 = table2.shape
    Bn = idx2.shape[0]
    NC, NS = 2, 16
    NW = NC * NS
    bpw = Bn // NW
    mesh = plsc.VectorSubcoreMesh(core_axis_name="c", subcore_axis_name="s")

    @functools.partial(
        pl.kernel,
        mesh=mesh,
        out_type=jax.ShapeDtypeStruct((Bn, D2), table2.dtype),
        scratch_types=[
            pltpu.VMEM((bpw,), jnp.int32),
            pltpu.VMEM((bpw, D2), jnp.float32),
            pltpu.SemaphoreType.DMA,
        ],
        compiler_params=pltpu.CompilerParams(use_tc_tiling_on_sc=True),
    )
    def gk(table_hbm, idx_hbm, out_hbm, idx_v, rows_v, sem):
        wid = lax.axis_index("s") * NC + lax.axis_index("c")
        base = wid * bpw
        pltpu.sync_copy(idx_hbm.at[pl.ds(base, bpw)], idx_v)
        pltpu.async_copy(table_hbm.at[idx_v], rows_v, sem).wait()
        pltpu.sync_copy(rows_v, out_hbm.at[pl.ds(base, bpw)])

    return gk(table2, idx2)


_G = 8192
_NP = 7


def _tpad_body(a_ref, b_ref, out_ref):
    a = jnp.transpose(a_ref[...], (1, 0))
    bb = jnp.transpose(b_ref[...], (1, 0))
    out_ref[...] = jnp.concatenate([a, bb], axis=1)


def _transpose_pack_table(table_t, V):
    D, _ = table_t.shape
    nblk = pl.cdiv(V, _G)
    return pl.pallas_call(
        _tpad_body,
        grid=(_NP,),
        in_specs=[
            pl.BlockSpec((D, _G), lambda i: (0, i)),
            pl.BlockSpec((D, _G), lambda i: (0, jnp.minimum(i + _NP, nblk - 1))),
        ],
        out_specs=pl.BlockSpec((_G, 2 * D), lambda i: (i, 0)),
        out_shape=jax.ShapeDtypeStruct((_NP * _G, 2 * D), table_t.dtype),
    )(table_t, table_t)


def _stats_body(nt, ds, slot_ref, meta_ref, gamma_ref, beta_ref, wt_ref,
                b_ref, scale_out, mnou_out, acc_ref):
    i = pl.program_id(0)
    n = pl.num_programs(0)

    @pl.when(i == 0)
    def _init():
        acc_ref[...] = jnp.zeros_like(acc_ref)

    x = slot_ref[...]
    acc_ref[0:ds, :] += jnp.sum(x, axis=0)
    acc_ref[ds:, :] += jnp.sum(x * x, axis=0)

    @pl.when(i == n - 1)
    def _finalize():
        meta = meta_ref[...]
        bsz = meta.shape[1]
        gam = gamma_ref[...]
        bet = beta_ref[...]
        wt = wt_ref[...]

        s1 = jnp.sum(acc_ref[0:ds, :], axis=1, keepdims=True)
        s2 = jnp.sum(acc_ref[ds:, :], axis=1, keepdims=True)
        mean_s = s1 / nt
        var_s = s2 / nt - mean_s * mean_s
        mean_m = jnp.sum(meta, axis=1, keepdims=True) / bsz
        var_m = (jnp.sum(meta * meta, axis=1, keepdims=True) / bsz
                 - mean_m * mean_m)

        scale_s = gam[0:ds, :] * lax.rsqrt(var_s + 1e-3)
        scale_m = gam[ds:, :] * lax.rsqrt(var_m + 1e-3)
        shift_s = bet[0:ds, :] - mean_s * scale_s
        shift_m = bet[ds:, :] - mean_m * scale_m

        bsum = (
            jnp.dot(wt[:, 0:ds], shift_s, preferred_element_type=jnp.float32)
            + jnp.dot(wt[:, ds:], shift_m, preferred_element_type=jnp.float32)
            + b_ref[...]
        )
        mm = (meta * scale_m).astype(jnp.bfloat16)
        wmt = wt[:, ds:].astype(jnp.bfloat16)
        mnou_out[...] = (
            jnp.dot(wmt, mm, preferred_element_type=jnp.float32) + bsum
        )
        scale_out[...] = scale_s


def _main_body(tb, ds, n_cls, slot_ref, rank_ref, mnou_ref, u_ref, scale_ref,
               wt_ref, rtt_ref, out_ref):
    bsz = slot_ref.shape[2]
    x3 = slot_ref[...]
    xs3 = (x3 * scale_ref[...]).astype(jnp.bfloat16)
    wst = wt_ref[...][:, 0:ds].astype(jnp.bfloat16)
    rtt = rtt_ref[...]
    idx2 = rank_ref[...]
    iot = lax.broadcasted_iota(jnp.int32, (n_cls, bsz), 0)
    mu = mnou_ref[...] + u_ref[...]

    wcat = jnp.concatenate([wst, rtt], axis=1)
    for k in range(tb):
        oh = (jnp.broadcast_to(idx2[k : k + 1, :], (n_cls, bsz)) == iot)
        a = jnp.concatenate([xs3[k], oh.astype(jnp.bfloat16)], axis=0)
        out_ref[k] = (
            jnp.dot(wcat, a, preferred_element_type=jnp.float32) + mu
        )


def kernel(slot_feat, meta_feat, uvcc, rank, uvcc_table, rank_table, gamma,
           beta, W, b):
    B, T, DS = slot_feat.shape
    DM = meta_feat.shape[1]
    D = W.shape[1]
    f32 = jnp.float32

    slot_t = slot_feat.transpose(1, 2, 0)
    meta_t = meta_feat.transpose(1, 0)
    rank_t = rank.astype(jnp.int32).transpose(1, 0)
    wt = W.transpose(1, 0)
    gamma2 = gamma.reshape(DS + DM, 1).astype(f32)
    beta2 = beta.reshape(DS + DM, 1).astype(f32)
    b2 = b.reshape(D, 1).astype(f32)
    n_cls = 208
    rtt_bf = (
        jnp.zeros((D, n_cls), jnp.bfloat16)
        .at[:, : rank_table.shape[0]]
        .set(rank_table.transpose(1, 0).astype(jnp.bfloat16))
    )

    V = uvcc_table.shape[0]
    P = _NP * _G
    tab_pack = _transpose_pack_table(uvcc_table.astype(f32).transpose(1, 0), V)
    uv = uvcc.astype(jnp.int32)
    u2 = _sc_gather(tab_pack, jnp.where(uv < P, uv, uv - P))
    u = jnp.where((uv < P)[:, None], u2[:, :D], u2[:, D:])
    u_t = u.transpose(1, 0)

    scale_s, mnou_t = pl.pallas_call(
        functools.partial(_stats_body, float(B * T), DS),
        grid=(T // _TB,),
        in_specs=[
            pl.BlockSpec((_TB, DS, B), lambda i: (i, 0, 0)),
            pl.BlockSpec((DM, B), lambda i: (0, 0)),
            pl.BlockSpec((DS + DM, 1), lambda i: (0, 0)),
            pl.BlockSpec((DS + DM, 1), lambda i: (0, 0)),
            pl.BlockSpec((D, DS + DM), lambda i: (0, 0)),
            pl.BlockSpec((D, 1), lambda i: (0, 0)),
        ],
        out_specs=[
            pl.BlockSpec((DS, 1), lambda i: (0, 0)),
            pl.BlockSpec((D, B), lambda i: (0, 0)),
        ],
        out_shape=[
            jax.ShapeDtypeStruct((DS, 1), f32),
            jax.ShapeDtypeStruct((D, B), f32),
        ],
        scratch_shapes=[pltpu.VMEM((2 * DS, B), f32)],
    )(slot_t, meta_t, gamma2, beta2, wt, b2)

    out_t = pl.pallas_call(
        functools.partial(_main_body, _TB, DS, n_cls),
        grid=(T // _TB,),
        in_specs=[
            pl.BlockSpec((_TB, DS, B), lambda i: (i, 0, 0)),
            pl.BlockSpec((_TB, B), lambda i: (i, 0)),
            pl.BlockSpec((D, B), lambda i: (0, 0)),
            pl.BlockSpec((D, B), lambda i: (0, 0)),
            pl.BlockSpec((DS, 1), lambda i: (0, 0)),
            pl.BlockSpec((D, DS + DM), lambda i: (0, 0)),
            pl.BlockSpec((D, n_cls), lambda i: (0, 0)),
        ],
        out_specs=pl.BlockSpec((_TB, D, B), lambda i: (i, 0, 0)),
        out_shape=jax.ShapeDtypeStruct((T, D, B), f32),
    )(slot_t, rank_t, mnou_t, u_t, scale_s, wt, rtt_bf)

    return out_t.transpose(2, 0, 1)


# speedup vs baseline: 1.0145x; 1.0145x over previous
import functools

import jax
import jax.numpy as jnp
from jax import lax
from jax.experimental import pallas as pl
from jax.experimental.pallas import tpu as pltpu
from jax.experimental.pallas import tpu_sc as plsc

_TB = 40


def _sc_gather(table2, idx2):
    V2, D2 = table2.shape
    Bn = idx2.shape[0]
    NC, NS = 2, 16
    NW = NC * NS
    bpw = Bn // NW
    mesh = plsc.VectorSubcoreMesh(core_axis_name="c", subcore_axis_name="s")

    @functools.partial(
        pl.kernel,
        mesh=mesh,
        out_type=jax.ShapeDtypeStruct((Bn, D2), table2.dtype),
        scratch_types=[
            pltpu.VMEM((bpw,), jnp.int32),
            pltpu.VMEM((bpw, D2), jnp.float32),
            pltpu.SemaphoreType.DMA,
        ],
        compiler_params=pltpu.CompilerParams(use_tc_tiling_on_sc=True),
    )
    def gk(table_hbm, idx_hbm, out_hbm, idx_v, rows_v, sem):
        wid = lax.axis_index("s") * NC + lax.axis_index("c")
        base = wid * bpw
        pltpu.sync_copy(idx_hbm.at[pl.ds(base, bpw)], idx_v)
        pltpu.async_copy(table_hbm.at[idx_v], rows_v, sem).wait()
        pltpu.sync_copy(rows_v, out_hbm.at[pl.ds(base, bpw)])

    return gk(table2, idx2)


def _tpad_body(g, slab_ref, out_ref):
    x = slab_ref[...]
    y = jnp.transpose(x, (1, 0))
    z = jnp.zeros((g, 128 - y.shape[1]), y.dtype)
    out_ref[...] = jnp.concatenate([y, z], axis=1)


def _transpose_pad_table(table_t, V):
    D, _ = table_t.shape
    g = 8192
    return pl.pallas_call(
        functools.partial(_tpad_body, g),
        grid=(pl.cdiv(V, g),),
        in_specs=[pl.BlockSpec((D, g), lambda i: (0, i))],
        out_specs=pl.BlockSpec((g, 128), lambda i: (i, 0)),
        out_shape=jax.ShapeDtypeStruct((V, 128), table_t.dtype),
    )(table_t)


def _stats_body(nt, ds, slot_ref, meta_ref, gamma_ref, beta_ref, wt_ref,
                b_ref, scale_out, mnou_out, acc_ref):
    i = pl.program_id(0)
    n = pl.num_programs(0)

    @pl.when(i == 0)
    def _init():
        acc_ref[...] = jnp.zeros_like(acc_ref)

    x = slot_ref[...]
    acc_ref[0:ds, :] += jnp.sum(x, axis=0)
    acc_ref[ds:, :] += jnp.sum(x * x, axis=0)

    @pl.when(i == n - 1)
    def _finalize():
        meta = meta_ref[...]
        bsz = meta.shape[1]
        gam = gamma_ref[...]
        bet = beta_ref[...]
        wt = wt_ref[...]

        s1 = jnp.sum(acc_ref[0:ds, :], axis=1, keepdims=True)
        s2 = jnp.sum(acc_ref[ds:, :], axis=1, keepdims=True)
        mean_s = s1 / nt
        var_s = s2 / nt - mean_s * mean_s
        mean_m = jnp.sum(meta, axis=1, keepdims=True) / bsz
        var_m = (jnp.sum(meta * meta, axis=1, keepdims=True) / bsz
                 - mean_m * mean_m)

        scale_s = gam[0:ds, :] * lax.rsqrt(var_s + 1e-3)
        scale_m = gam[ds:, :] * lax.rsqrt(var_m + 1e-3)
        shift_s = bet[0:ds, :] - mean_s * scale_s
        shift_m = bet[ds:, :] - mean_m * scale_m

        bsum = (
            jnp.dot(wt[:, 0:ds], shift_s, preferred_element_type=jnp.float32)
            + jnp.dot(wt[:, ds:], shift_m, preferred_element_type=jnp.float32)
            + b_ref[...]
        )
        mm = (meta * scale_m).astype(jnp.bfloat16)
        wmt = wt[:, ds:].astype(jnp.bfloat16)
        mnou_out[...] = (
            jnp.dot(wmt, mm, preferred_element_type=jnp.float32) + bsum
        )
        scale_out[...] = scale_s


def _main_body(tb, ds, n_cls, slot_ref, rank_ref, mnou_ref, u_ref, scale_ref,
               wt_ref, rtt_ref, out_ref):
    bsz = slot_ref.shape[2]
    x3 = slot_ref[...]
    xs3 = (x3 * scale_ref[...]).astype(jnp.bfloat16)
    wst = wt_ref[...][:, 0:ds].astype(jnp.bfloat16)
    rtt = rtt_ref[...]
    idx2 = rank_ref[...]
    iot = lax.broadcasted_iota(jnp.int32, (n_cls, bsz), 0)
    mu = mnou_ref[...] + u_ref[...]

    wcat = jnp.concatenate([wst, rtt], axis=1)
    for k in range(tb):
        oh = (jnp.broadcast_to(idx2[k : k + 1, :], (n_cls, bsz)) == iot)
        a = jnp.concatenate([xs3[k], oh.astype(jnp.bfloat16)], axis=0)
        out_ref[k] = (
            jnp.dot(wcat, a, preferred_element_type=jnp.float32) + mu
        )


def kernel(slot_feat, meta_feat, uvcc, rank, uvcc_table, rank_table, gamma,
           beta, W, b):
    B, T, DS = slot_feat.shape
    DM = meta_feat.shape[1]
    D = W.shape[1]
    f32 = jnp.float32

    slot_t = slot_feat.transpose(1, 2, 0)
    meta_t = meta_feat.transpose(1, 0)
    rank_t = rank.astype(jnp.int32).transpose(1, 0)
    wt = W.transpose(1, 0)
    gamma2 = gamma.reshape(DS + DM, 1).astype(f32)
    beta2 = beta.reshape(DS + DM, 1).astype(f32)
    b2 = b.reshape(D, 1).astype(f32)
    n_cls = 208
    rtt_bf = (
        jnp.zeros((D, n_cls), jnp.bfloat16)
        .at[:, : rank_table.shape[0]]
        .set(rank_table.transpose(1, 0).astype(jnp.bfloat16))
    )

    V = uvcc_table.shape[0]
    tab_pad = _transpose_pad_table(uvcc_table.astype(f32).transpose(1, 0), V)
    u2 = _sc_gather(tab_pad, uvcc.astype(jnp.int32))
    u = u2[:, :D]
    u_t = u.transpose(1, 0)

    scale_s, mnou_t = pl.pallas_call(
        functools.partial(_stats_body, float(B * T), DS),
        grid=(T // _TB,),
        in_specs=[
            pl.BlockSpec((_TB, DS, B), lambda i: (i, 0, 0)),
            pl.BlockSpec((DM, B), lambda i: (0, 0)),
            pl.BlockSpec((DS + DM, 1), lambda i: (0, 0)),
            pl.BlockSpec((DS + DM, 1), lambda i: (0, 0)),
            pl.BlockSpec((D, DS + DM), lambda i: (0, 0)),
            pl.BlockSpec((D, 1), lambda i: (0, 0)),
        ],
        out_specs=[
            pl.BlockSpec((DS, 1), lambda i: (0, 0)),
            pl.BlockSpec((D, B), lambda i: (0, 0)),
        ],
        out_shape=[
            jax.ShapeDtypeStruct((DS, 1), f32),
            jax.ShapeDtypeStruct((D, B), f32),
        ],
        scratch_shapes=[pltpu.VMEM((2 * DS, B), f32)],
    )(slot_t, meta_t, gamma2, beta2, wt, b2)

    out_t = pl.pallas_call(
        functools.partial(_main_body, _TB, DS, n_cls),
        grid=(T // _TB,),
        in_specs=[
            pl.BlockSpec((_TB, DS, B), lambda i: (i, 0, 0)),
            pl.BlockSpec((_TB, B), lambda i: (i, 0)),
            pl.BlockSpec((D, B), lambda i: (0, 0)),
            pl.BlockSpec((D, B), lambda i: (0, 0)),
            pl.BlockSpec((DS, 1), lambda i: (0, 0)),
            pl.BlockSpec((D, DS + DM), lambda i: (0, 0)),
            pl.BlockSpec((D, n_cls), lambda i: (0, 0)),
        ],
        out_specs=pl.BlockSpec((_TB, D, B), lambda i: (i, 0, 0)),
        out_shape=jax.ShapeDtypeStruct((T, D, B), f32),
    )(slot_t, rank_t, mnou_t, u_t, scale_s, wt, rtt_bf)

    return out_t.transpose(2, 0, 1)
